# baseline (speedup 1.0000x reference)
import functools

import jax
import jax.numpy as jnp
from jax import lax
from jax.experimental import pallas as pl
from jax.experimental.pallas import tpu as pltpu
from jax.experimental.pallas import tpu_sc as plsc

_B = 1024
_S = 200
_EMB = 128
_POS_V = 50
_NER_V = 20
_IN = 192
_BB = 8
_NW = 32
_CH = 128


def _sc_gather(table, idx_3d, d):
    nchunk = idx_3d.shape[1]
    tot = _NW * nchunk * _CH
    rows_per_w = nchunk * _CH
    mesh = plsc.VectorSubcoreMesh(core_axis_name="c", subcore_axis_name="s")

    @functools.partial(
        pl.kernel,
        out_type=jax.ShapeDtypeStruct((tot, d), jnp.float32),
        mesh=mesh,
        scratch_types=[
            pltpu.VMEM((nchunk, _CH), jnp.int32),
            pltpu.VMEM((_CH, d), jnp.float32),
            pltpu.VMEM((_CH, d), jnp.float32),
            pltpu.SemaphoreType.DMA,
            pltpu.SemaphoreType.DMA,
        ],
    )
    def gather_kernel(table_hbm, idx_hbm, out_hbm, idx_v, rows0, rows1, sem0, sem1):
        wid = lax.axis_index("s") * 2 + lax.axis_index("c")
        base = wid * rows_per_w
        pltpu.sync_copy(idx_hbm.at[wid], idx_v)
        pltpu.async_copy(table_hbm.at[idx_v.at[0]], rows0, sem0)

        def body(j, carry):
            buf = lax.rem(j, 2)

            @pl.when(j + 1 < nchunk)
            def _():
                @pl.when(buf == 0)
                def _():
                    pltpu.async_copy(table_hbm.at[idx_v.at[j + 1]], rows1, sem1)

                @pl.when(buf == 1)
                def _():
                    pltpu.async_copy(table_hbm.at[idx_v.at[j + 1]], rows0, sem0)

            @pl.when(buf == 0)
            def _():
                pltpu.make_async_copy(table_hbm.at[idx_v.at[0]], rows0, sem0).wait()
                pltpu.sync_copy(rows0, out_hbm.at[pl.ds(base + j * _CH, _CH)])

            @pl.when(buf == 1)
            def _():
                pltpu.make_async_copy(table_hbm.at[idx_v.at[0]], rows1, sem1).wait()
                pltpu.sync_copy(rows1, out_hbm.at[pl.ds(base + j * _CH, _CH)])

            return carry

        lax.fori_loop(0, nchunk, body, 0)

    return gather_kernel(table, idx_3d)


def _tc_body(w_ref, pos_ref, ner_ref, fused_ref, posemb_ref, g_ref, b_ref,
             emb_out):
    nv = _POS_V + _NER_V
    w = w_ref[...]
    p = pos_ref[..., 0][..., None]
    n = ner_ref[..., 0][..., None]
    io = lax.broadcasted_iota(jnp.int32, (_BB, _S, nv), 2)
    oh = ((io == p) | (io == n + _POS_V)).astype(jnp.float32)
    pn = lax.dot_general(oh, fused_ref[...],
                         dimension_numbers=(((2,), (0,)), ((), ())),
                         preferred_element_type=jnp.float32)
    e = jnp.concatenate([w, pn], axis=2)
    e = e + posemb_ref[:_S][None, :, :]
    mean = jnp.mean(e, axis=-1, keepdims=True)
    var = jnp.mean((e - mean) * (e - mean), axis=-1, keepdims=True)
    emb_out[...] = (e - mean) * lax.rsqrt(var + 1e-5) * g_ref[...] + b_ref[...]


_SI = 8
_BL = 256


def _mask_body(headt_ref, headti_ref, maskt_ref, dep_out, pad_out, seq_out,
               adj_out):
    i0 = pl.program_id(0) * _SI
    head_all = headt_ref[...]
    h_all = jnp.clip(head_all - 1, 0, _S - 1)
    valid_all = head_all > 0
    ii1 = i0 + lax.broadcasted_iota(jnp.int32, (_SI, _S, 1), 0)
    jj1 = lax.broadcasted_iota(jnp.int32, (_SI, _S, 1), 1)
    head_i = headti_ref[...]
    h_i = jnp.clip(head_i - 1, 0, _S - 1)[:, None, :]
    v_i = (head_i > 0)[:, None, :]
    c1 = (h_i == jj1) & v_i
    c2 = (h_all[None, :, :] == ii1) & valid_all[None, :, :]
    a = c1 | c2 | (ii1 == jj1)
    adj_out[...] = a.astype(jnp.float32)
    dep_out[...] = (~a).astype(jnp.int8)
    lrow = jnp.sum((maskt_ref[...] == 0).astype(jnp.int32), axis=0,
                   keepdims=True)
    pad = ~(lax.broadcasted_iota(jnp.int32, (_S, _BL), 0) < lrow)
    pad_out[...] = jnp.broadcast_to(pad.astype(jnp.int8)[None, :, :],
                                    (_SI, _S, _BL))
    seq_out[...] = jnp.broadcast_to((~(jj1 <= ii1)).astype(jnp.int8),
                                    (_SI, _S, _BL))


_SS = 5
_EBL = 512


def _emb_body(w_ref, post_ref, nert_ref, fusedt_ref, posemb_ref, g_ref, b_ref,
              emb_out):
    nv = _POS_V + _NER_V
    io = lax.broadcasted_iota(jnp.int32, (nv, _EBL), 0)
    for k in range(_SS):
        w_t = jnp.transpose(w_ref[k], (1, 0))
        p = post_ref[k]
        n = nert_ref[k]
        oh = ((io == p) | (io == n + _POS_V)).astype(jnp.float32)
        pn = jnp.dot(fusedt_ref[...], oh,
                     preferred_element_type=jnp.float32)
        e = jnp.concatenate([w_t, pn], axis=0) + posemb_ref[k]
        mean = jnp.mean(e, axis=0, keepdims=True)
        var = jnp.mean((e - mean) * (e - mean), axis=0, keepdims=True)
        emb_out[k] = (e - mean) * lax.rsqrt(var + 1e-5) * g_ref[...] + b_ref[...]


def kernel(words, masks, pos, ner, deprel, head, subj_pos, obj_pos,
           subj_type, obj_type, emb, pos_emb, ner_emb, position_emb,
           ln_gamma, ln_beta):
    nchunk = _B * _S // (_NW * _CH)
    words_t3d = words.T.reshape(_NW, nchunk, _CH)
    word_rows_t = _sc_gather(emb, words_t3d, _EMB).reshape(_S, _B, _EMB)

    nv = _POS_V + _NER_V
    fused = jnp.zeros((nv, 64), jnp.float32)
    fused = fused.at[:_POS_V, :32].set(pos_emb)
    fused = fused.at[_POS_V:, 32:].set(ner_emb)

    f32 = jnp.float32
    i8 = jnp.int8
    embt = pl.pallas_call(
        _emb_body,
        grid=(_S // _SS, _B // _EBL),
        in_specs=[
            pl.BlockSpec((_SS, _EBL, _EMB), lambda s, b: (s, b, 0)),
            pl.BlockSpec((_SS, 1, _EBL), lambda s, b: (s, 0, b)),
            pl.BlockSpec((_SS, 1, _EBL), lambda s, b: (s, 0, b)),
            pl.BlockSpec((64, nv), lambda s, b: (0, 0)),
            pl.BlockSpec((_SS, _IN, 1), lambda s, b: (s, 0, 0)),
            pl.BlockSpec((_IN, 1), lambda s, b: (0, 0)),
            pl.BlockSpec((_IN, 1), lambda s, b: (0, 0)),
        ],
        out_specs=pl.BlockSpec((_SS, _IN, _EBL), lambda s, b: (s, 0, b)),
        out_shape=jax.ShapeDtypeStruct((_S, _IN, _B), f32),
    )(word_rows_t, pos.T[:, None, :], ner.T[:, None, :], fused.T,
      position_emb[:_S][:, :, None], ln_gamma.reshape(_IN, 1),
      ln_beta.reshape(_IN, 1))
    embeddings = jnp.transpose(embt, (2, 0, 1))

    head_t = head.T
    masks_t = masks.T
    colspec = pl.BlockSpec((_S, _BL), lambda i, b: (0, b))
    rowspec = pl.BlockSpec((_SI, _BL), lambda i, b: (i, b))
    outt = lambda dt: pl.BlockSpec((_SI, _S, _BL), lambda i, b: (i, 0, b))
    ssb = lambda dt: jax.ShapeDtypeStruct((_S, _S, _B), dt)
    dep_t, pad_t, seq_t, adj_t = pl.pallas_call(
        _mask_body,
        grid=(_S // _SI, _B // _BL),
        in_specs=[colspec, rowspec, colspec],
        out_specs=(outt(i8), outt(i8), outt(i8), outt(f32)),
        out_shape=(ssb(i8), ssb(i8), ssb(i8), ssb(f32)),
    )(head_t, head_t, masks_t)
    tr = lambda x: jnp.transpose(x, (2, 0, 1))
    trb = lambda x: jnp.transpose(x.astype(jnp.bool_), (2, 0, 1))
    return (embeddings, trb(dep_t), trb(pad_t), trb(seq_t), tr(adj_t))


# speedup vs baseline: 1.1356x; 1.1356x over previous
import functools

import jax
import jax.numpy as jnp
from jax import lax
from jax.experimental import pallas as pl
from jax.experimental.pallas import tpu as pltpu
from jax.experimental.pallas import tpu_sc as plsc

_B = 1024
_S = 200
_EMB = 128
_POS_V = 50
_NER_V = 20
_IN = 192
_BB = 8
_NW = 32
_CH = 128


def _sc_gather(table, idx_3d, d):
    nchunk = idx_3d.shape[1]
    tot = _NW * nchunk * _CH
    rows_per_w = nchunk * _CH
    mesh = plsc.VectorSubcoreMesh(core_axis_name="c", subcore_axis_name="s")

    @functools.partial(
        pl.kernel,
        out_type=jax.ShapeDtypeStruct((tot, d), jnp.float32),
        mesh=mesh,
        scratch_types=[
            pltpu.VMEM((nchunk, _CH), jnp.int32),
            pltpu.VMEM((_CH, d), jnp.float32),
            pltpu.VMEM((_CH, d), jnp.float32),
            pltpu.SemaphoreType.DMA,
            pltpu.SemaphoreType.DMA,
        ],
    )
    def gather_kernel(table_hbm, idx_hbm, out_hbm, idx_v, rows0, rows1, sem0, sem1):
        wid = lax.axis_index("s") * 2 + lax.axis_index("c")
        base = wid * rows_per_w
        pltpu.sync_copy(idx_hbm.at[wid], idx_v)
        pltpu.async_copy(table_hbm.at[idx_v.at[0]], rows0, sem0)

        def body(j, carry):
            buf = lax.rem(j, 2)

            @pl.when(j + 1 < nchunk)
            def _():
                @pl.when(buf == 0)
                def _():
                    pltpu.async_copy(table_hbm.at[idx_v.at[j + 1]], rows1, sem1)

                @pl.when(buf == 1)
                def _():
                    pltpu.async_copy(table_hbm.at[idx_v.at[j + 1]], rows0, sem0)

            @pl.when(buf == 0)
            def _():
                pltpu.make_async_copy(table_hbm.at[idx_v.at[0]], rows0, sem0).wait()
                pltpu.sync_copy(rows0, out_hbm.at[pl.ds(base + j * _CH, _CH)])

            @pl.when(buf == 1)
            def _():
                pltpu.make_async_copy(table_hbm.at[idx_v.at[0]], rows1, sem1).wait()
                pltpu.sync_copy(rows1, out_hbm.at[pl.ds(base + j * _CH, _CH)])

            return carry

        lax.fori_loop(0, nchunk, body, 0)

    return gather_kernel(table, idx_3d)


def _tc_body(w_ref, pos_ref, ner_ref, fused_ref, posemb_ref, g_ref, b_ref,
             emb_out):
    nv = _POS_V + _NER_V
    w = w_ref[...]
    p = pos_ref[..., 0][..., None]
    n = ner_ref[..., 0][..., None]
    io = lax.broadcasted_iota(jnp.int32, (_BB, _S, nv), 2)
    oh = ((io == p) | (io == n + _POS_V)).astype(jnp.float32)
    pn = lax.dot_general(oh, fused_ref[...],
                         dimension_numbers=(((2,), (0,)), ((), ())),
                         preferred_element_type=jnp.float32)
    e = jnp.concatenate([w, pn], axis=2)
    e = e + posemb_ref[:_S][None, :, :]
    mean = jnp.mean(e, axis=-1, keepdims=True)
    var = jnp.mean((e - mean) * (e - mean), axis=-1, keepdims=True)
    emb_out[...] = (e - mean) * lax.rsqrt(var + 1e-5) * g_ref[...] + b_ref[...]


_SI = 8
_BL = 512


def _mask_body(headt_ref, headti_ref, maskt_ref, dep_out, pad_out, seq_out,
               adj_out):
    i0 = pl.program_id(0) * _SI
    head_all = headt_ref[...]
    h_all = jnp.clip(head_all - 1, 0, _S - 1)
    valid_all = head_all > 0
    ii = i0 + lax.broadcasted_iota(jnp.int32, (_SI, _S, _BL), 0)
    jj = lax.broadcasted_iota(jnp.int32, (_SI, _S, _BL), 1)
    head_i = headti_ref[...]
    h_i = jnp.clip(head_i - 1, 0, _S - 1)[:, None, :]
    v_i = (head_i > 0)[:, None, :]
    c1 = (h_i == jj) & v_i
    c2 = (h_all[None, :, :] == ii) & valid_all[None, :, :]
    a = c1 | c2 | (ii == jj)
    adj_out[...] = a.astype(jnp.float32)
    dep_out[...] = (~a).astype(jnp.int8)
    lrow = jnp.sum((maskt_ref[...] == 0).astype(jnp.int32), axis=0,
                   keepdims=True)
    pad = ~(lax.broadcasted_iota(jnp.int32, (_S, _BL), 0) < lrow)
    pad_out[...] = jnp.broadcast_to(pad.astype(jnp.int8)[None, :, :],
                                    (_SI, _S, _BL))
    seq_out[...] = (~(jj <= ii)).astype(jnp.int8)


_SS = 5
_EBL = 1024


def _emb_body(w_ref, post_ref, nert_ref, fusedt_ref, posemb_ref, g_ref, b_ref,
              emb_out):
    nv = _POS_V + _NER_V
    io = lax.broadcasted_iota(jnp.int32, (nv, _EBL), 0)
    for k in range(_SS):
        w_t = jnp.transpose(w_ref[k], (1, 0))
        p = post_ref[k]
        n = nert_ref[k]
        oh = ((io == p) | (io == n + _POS_V)).astype(jnp.float32)
        pn = jnp.dot(fusedt_ref[...], oh,
                     preferred_element_type=jnp.float32)
        e = jnp.concatenate([w_t, pn], axis=0) + posemb_ref[k]
        mean = jnp.mean(e, axis=0, keepdims=True)
        var = jnp.mean((e - mean) * (e - mean), axis=0, keepdims=True)
        emb_out[k] = (e - mean) * lax.rsqrt(var + 1e-5) * g_ref[...] + b_ref[...]


def kernel(words, masks, pos, ner, deprel, head, subj_pos, obj_pos,
           subj_type, obj_type, emb, pos_emb, ner_emb, position_emb,
           ln_gamma, ln_beta):
    nchunk = _B * _S // (_NW * _CH)
    words_t3d = words.T.reshape(_NW, nchunk, _CH)
    word_rows_t = _sc_gather(emb, words_t3d, _EMB).reshape(_S, _B, _EMB)

    nv = _POS_V + _NER_V
    fused = jnp.zeros((nv, 64), jnp.float32)
    fused = fused.at[:_POS_V, :32].set(pos_emb)
    fused = fused.at[_POS_V:, 32:].set(ner_emb)

    f32 = jnp.float32
    i8 = jnp.int8
    embt = pl.pallas_call(
        _emb_body,
        grid=(_S // _SS, _B // _EBL),
        in_specs=[
            pl.BlockSpec((_SS, _EBL, _EMB), lambda s, b: (s, b, 0)),
            pl.BlockSpec((_SS, 1, _EBL), lambda s, b: (s, 0, b)),
            pl.BlockSpec((_SS, 1, _EBL), lambda s, b: (s, 0, b)),
            pl.BlockSpec((64, nv), lambda s, b: (0, 0)),
            pl.BlockSpec((_SS, _IN, 1), lambda s, b: (s, 0, 0)),
            pl.BlockSpec((_IN, 1), lambda s, b: (0, 0)),
            pl.BlockSpec((_IN, 1), lambda s, b: (0, 0)),
        ],
        out_specs=pl.BlockSpec((_SS, _IN, _EBL), lambda s, b: (s, 0, b)),
        out_shape=jax.ShapeDtypeStruct((_S, _IN, _B), f32),
    )(word_rows_t, pos.T[:, None, :], ner.T[:, None, :], fused.T,
      position_emb[:_S][:, :, None], ln_gamma.reshape(_IN, 1),
      ln_beta.reshape(_IN, 1))
    embeddings = jnp.transpose(embt, (2, 0, 1))

    head_t = head.T
    masks_t = masks.T
    colspec = pl.BlockSpec((_S, _BL), lambda i, b: (0, b))
    rowspec = pl.BlockSpec((_SI, _BL), lambda i, b: (i, b))
    outt = lambda dt: pl.BlockSpec((_SI, _S, _BL), lambda i, b: (i, 0, b))
    ssb = lambda dt: jax.ShapeDtypeStruct((_S, _S, _B), dt)
    dep_t, pad_t, seq_t, adj_t = pl.pallas_call(
        _mask_body,
        grid=(_S // _SI, _B // _BL),
        in_specs=[colspec, rowspec, colspec],
        out_specs=(outt(i8), outt(i8), outt(i8), outt(f32)),
        out_shape=(ssb(i8), ssb(i8), ssb(i8), ssb(f32)),
    )(head_t, head_t, masks_t)
    tr = lambda x: jnp.transpose(x, (2, 0, 1))
    trb = lambda x: jnp.transpose(x.astype(jnp.bool_), (2, 0, 1))
    return (embeddings, trb(dep_t), trb(pad_t), trb(seq_t), tr(adj_t))


# speedup vs baseline: 1.1826x; 1.0414x over previous
import functools

import jax
import jax.numpy as jnp
from jax import lax
from jax.experimental import pallas as pl
from jax.experimental.pallas import tpu as pltpu
from jax.experimental.pallas import tpu_sc as plsc

_B = 1024
_S = 200
_EMB = 128
_POS_V = 50
_NER_V = 20
_IN = 192
_BB = 8
_NW = 32
_CH = 128


def _sc_gather(table, idx_3d, d):
    nchunk = idx_3d.shape[1]
    tot = _NW * nchunk * _CH
    rows_per_w = nchunk * _CH
    mesh = plsc.VectorSubcoreMesh(core_axis_name="c", subcore_axis_name="s")

    @functools.partial(
        pl.kernel,
        out_type=jax.ShapeDtypeStruct((tot, d), jnp.float32),
        mesh=mesh,
        scratch_types=[
            pltpu.VMEM((nchunk, _CH), jnp.int32),
            pltpu.VMEM((_CH, d), jnp.float32),
            pltpu.VMEM((_CH, d), jnp.float32),
            pltpu.SemaphoreType.DMA,
            pltpu.SemaphoreType.DMA,
        ],
    )
    def gather_kernel(table_hbm, idx_hbm, out_hbm, idx_v, rows0, rows1, sem0, sem1):
        wid = lax.axis_index("s") * 2 + lax.axis_index("c")
        base = wid * rows_per_w
        pltpu.sync_copy(idx_hbm.at[wid], idx_v)
        pltpu.async_copy(table_hbm.at[idx_v.at[0]], rows0, sem0)

        def body(j, carry):
            buf = lax.rem(j, 2)

            @pl.when(j + 1 < nchunk)
            def _():
                @pl.when(buf == 0)
                def _():
                    pltpu.async_copy(table_hbm.at[idx_v.at[j + 1]], rows1, sem1)

                @pl.when(buf == 1)
                def _():
                    pltpu.async_copy(table_hbm.at[idx_v.at[j + 1]], rows0, sem0)

            @pl.when(buf == 0)
            def _():
                pltpu.make_async_copy(table_hbm.at[idx_v.at[0]], rows0, sem0).wait()
                pltpu.sync_copy(rows0, out_hbm.at[pl.ds(base + j * _CH, _CH)])

            @pl.when(buf == 1)
            def _():
                pltpu.make_async_copy(table_hbm.at[idx_v.at[0]], rows1, sem1).wait()
                pltpu.sync_copy(rows1, out_hbm.at[pl.ds(base + j * _CH, _CH)])

            return carry

        lax.fori_loop(0, nchunk, body, 0)

    return gather_kernel(table, idx_3d)


def _tc_body(w_ref, pos_ref, ner_ref, fused_ref, posemb_ref, g_ref, b_ref,
             emb_out):
    nv = _POS_V + _NER_V
    w = w_ref[...]
    p = pos_ref[..., 0][..., None]
    n = ner_ref[..., 0][..., None]
    io = lax.broadcasted_iota(jnp.int32, (_BB, _S, nv), 2)
    oh = ((io == p) | (io == n + _POS_V)).astype(jnp.float32)
    pn = lax.dot_general(oh, fused_ref[...],
                         dimension_numbers=(((2,), (0,)), ((), ())),
                         preferred_element_type=jnp.float32)
    e = jnp.concatenate([w, pn], axis=2)
    e = e + posemb_ref[:_S][None, :, :]
    mean = jnp.mean(e, axis=-1, keepdims=True)
    var = jnp.mean((e - mean) * (e - mean), axis=-1, keepdims=True)
    emb_out[...] = (e - mean) * lax.rsqrt(var + 1e-5) * g_ref[...] + b_ref[...]


_SI = 8
_BL = 1024


def _mask_body(headt_ref, headti_ref, maskt_ref, dep_out, pad_out, seq_out,
               adj_out):
    i0 = pl.program_id(0) * _SI
    head_all = headt_ref[...]
    h_all = jnp.clip(head_all - 1, 0, _S - 1)
    valid_all = head_all > 0
    ii = i0 + lax.broadcasted_iota(jnp.int32, (_SI, _S, _BL), 0)
    jj = lax.broadcasted_iota(jnp.int32, (_SI, _S, _BL), 1)
    head_i = headti_ref[...]
    h_i = jnp.clip(head_i - 1, 0, _S - 1)[:, None, :]
    v_i = (head_i > 0)[:, None, :]
    c1 = (h_i == jj) & v_i
    c2 = (h_all[None, :, :] == ii) & valid_all[None, :, :]
    a = c1 | c2 | (ii == jj)
    adj_out[...] = a.astype(jnp.float32)
    dep_out[...] = (~a).astype(jnp.int8)
    lrow = jnp.sum((maskt_ref[...] == 0).astype(jnp.int32), axis=0,
                   keepdims=True)
    pad = ~(lax.broadcasted_iota(jnp.int32, (_S, _BL), 0) < lrow)
    pad_out[...] = jnp.broadcast_to(pad.astype(jnp.int8)[None, :, :],
                                    (_SI, _S, _BL))
    seq_out[...] = (~(jj <= ii)).astype(jnp.int8)


_SS = 10
_EBL = 1024


def _emb_body(w_ref, post_ref, nert_ref, fusedt_ref, posemb_ref, g_ref, b_ref,
              emb_out):
    nv = _POS_V + _NER_V
    io = lax.broadcasted_iota(jnp.int32, (nv, _EBL), 0)
    for k in range(_SS):
        w_t = jnp.transpose(w_ref[k], (1, 0))
        p = post_ref[k]
        n = nert_ref[k]
        oh = ((io == p) | (io == n + _POS_V)).astype(jnp.float32)
        pn = jnp.dot(fusedt_ref[...], oh,
                     preferred_element_type=jnp.float32)
        e = jnp.concatenate([w_t, pn], axis=0) + posemb_ref[k]
        mean = jnp.mean(e, axis=0, keepdims=True)
        var = jnp.mean((e - mean) * (e - mean), axis=0, keepdims=True)
        emb_out[k] = (e - mean) * lax.rsqrt(var + 1e-5) * g_ref[...] + b_ref[...]


def kernel(words, masks, pos, ner, deprel, head, subj_pos, obj_pos,
           subj_type, obj_type, emb, pos_emb, ner_emb, position_emb,
           ln_gamma, ln_beta):
    nchunk = _B * _S // (_NW * _CH)
    words_t3d = words.T.reshape(_NW, nchunk, _CH)
    word_rows_t = _sc_gather(emb, words_t3d, _EMB).reshape(_S, _B, _EMB)

    nv = _POS_V + _NER_V
    fused = jnp.zeros((nv, 64), jnp.float32)
    fused = fused.at[:_POS_V, :32].set(pos_emb)
    fused = fused.at[_POS_V:, 32:].set(ner_emb)

    f32 = jnp.float32
    i8 = jnp.int8
    embt = pl.pallas_call(
        _emb_body,
        grid=(_S // _SS, _B // _EBL),
        in_specs=[
            pl.BlockSpec((_SS, _EBL, _EMB), lambda s, b: (s, b, 0)),
            pl.BlockSpec((_SS, 1, _EBL), lambda s, b: (s, 0, b)),
            pl.BlockSpec((_SS, 1, _EBL), lambda s, b: (s, 0, b)),
            pl.BlockSpec((64, nv), lambda s, b: (0, 0)),
            pl.BlockSpec((_SS, _IN, 1), lambda s, b: (s, 0, 0)),
            pl.BlockSpec((_IN, 1), lambda s, b: (0, 0)),
            pl.BlockSpec((_IN, 1), lambda s, b: (0, 0)),
        ],
        out_specs=pl.BlockSpec((_SS, _IN, _EBL), lambda s, b: (s, 0, b)),
        out_shape=jax.ShapeDtypeStruct((_S, _IN, _B), f32),
    )(word_rows_t, pos.T[:, None, :], ner.T[:, None, :], fused.T,
      position_emb[:_S][:, :, None], ln_gamma.reshape(_IN, 1),
      ln_beta.reshape(_IN, 1))
    embeddings = jnp.transpose(embt, (2, 0, 1))

    head_t = head.T
    masks_t = masks.T
    colspec = pl.BlockSpec((_S, _BL), lambda i, b: (0, b))
    rowspec = pl.BlockSpec((_SI, _BL), lambda i, b: (i, b))
    outt = lambda dt: pl.BlockSpec((_SI, _S, _BL), lambda i, b: (i, 0, b))
    ssb = lambda dt: jax.ShapeDtypeStruct((_S, _S, _B), dt)
    dep_t, pad_t, seq_t, adj_t = pl.pallas_call(
        _mask_body,
        grid=(_S // _SI, _B // _BL),
        in_specs=[colspec, rowspec, colspec],
        out_specs=(outt(i8), outt(i8), outt(i8), outt(f32)),
        out_shape=(ssb(i8), ssb(i8), ssb(i8), ssb(f32)),
    )(head_t, head_t, masks_t)
    tr = lambda x: jnp.transpose(x, (2, 0, 1))
    trb = lambda x: jnp.transpose(x.astype(jnp.bool_), (2, 0, 1))
    return (embeddings, trb(dep_t), trb(pad_t), trb(seq_t), tr(adj_t))


# speedup vs baseline: 1.1828x; 1.0002x over previous
import functools

import jax
import jax.numpy as jnp
from jax import lax
from jax.experimental import pallas as pl
from jax.experimental.pallas import tpu as pltpu
from jax.experimental.pallas import tpu_sc as plsc

_B = 1024
_S = 200
_EMB = 128
_POS_V = 50
_NER_V = 20
_IN = 192
_BB = 8
_NW = 32
_CH = 128


def _sc_gather(table, idx_3d, d):
    nchunk = idx_3d.shape[1]
    tot = _NW * nchunk * _CH
    rows_per_w = nchunk * _CH
    mesh = plsc.VectorSubcoreMesh(core_axis_name="c", subcore_axis_name="s")

    @functools.partial(
        pl.kernel,
        out_type=jax.ShapeDtypeStruct((tot, d), jnp.float32),
        mesh=mesh,
        scratch_types=[
            pltpu.VMEM((nchunk, _CH), jnp.int32),
            pltpu.VMEM((_CH, d), jnp.float32),
            pltpu.VMEM((_CH, d), jnp.float32),
            pltpu.SemaphoreType.DMA,
            pltpu.SemaphoreType.DMA,
        ],
    )
    def gather_kernel(table_hbm, idx_hbm, out_hbm, idx_v, rows0, rows1, sem0, sem1):
        wid = lax.axis_index("s") * 2 + lax.axis_index("c")
        base = wid * rows_per_w
        pltpu.sync_copy(idx_hbm.at[wid], idx_v)
        pltpu.async_copy(table_hbm.at[idx_v.at[0]], rows0, sem0)

        def body(j, carry):
            buf = lax.rem(j, 2)

            @pl.when(j + 1 < nchunk)
            def _():
                @pl.when(buf == 0)
                def _():
                    pltpu.async_copy(table_hbm.at[idx_v.at[j + 1]], rows1, sem1)

                @pl.when(buf == 1)
                def _():
                    pltpu.async_copy(table_hbm.at[idx_v.at[j + 1]], rows0, sem0)

            @pl.when(buf == 0)
            def _():
                pltpu.make_async_copy(table_hbm.at[idx_v.at[0]], rows0, sem0).wait()
                pltpu.sync_copy(rows0, out_hbm.at[pl.ds(base + j * _CH, _CH)])

            @pl.when(buf == 1)
            def _():
                pltpu.make_async_copy(table_hbm.at[idx_v.at[0]], rows1, sem1).wait()
                pltpu.sync_copy(rows1, out_hbm.at[pl.ds(base + j * _CH, _CH)])

            return carry

        lax.fori_loop(0, nchunk, body, 0)

    return gather_kernel(table, idx_3d)


def _tc_body(w_ref, pos_ref, ner_ref, fused_ref, posemb_ref, g_ref, b_ref,
             emb_out):
    nv = _POS_V + _NER_V
    w = w_ref[...]
    p = pos_ref[..., 0][..., None]
    n = ner_ref[..., 0][..., None]
    io = lax.broadcasted_iota(jnp.int32, (_BB, _S, nv), 2)
    oh = ((io == p) | (io == n + _POS_V)).astype(jnp.float32)
    pn = lax.dot_general(oh, fused_ref[...],
                         dimension_numbers=(((2,), (0,)), ((), ())),
                         preferred_element_type=jnp.float32)
    e = jnp.concatenate([w, pn], axis=2)
    e = e + posemb_ref[:_S][None, :, :]
    mean = jnp.mean(e, axis=-1, keepdims=True)
    var = jnp.mean((e - mean) * (e - mean), axis=-1, keepdims=True)
    emb_out[...] = (e - mean) * lax.rsqrt(var + 1e-5) * g_ref[...] + b_ref[...]


_SI = 8
_BL = 1024


def _mask_body(headt_ref, headti_ref, maskt_ref, dep_out, pad_out, seq_out,
               adj_out):
    i0 = pl.program_id(0) * _SI
    head_all = headt_ref[...]
    h_all = jnp.clip(head_all - 1, 0, _S - 1)
    valid_all = head_all > 0
    ii = i0 + lax.broadcasted_iota(jnp.int32, (_SI, _S, _BL), 0)
    jj = lax.broadcasted_iota(jnp.int32, (_SI, _S, _BL), 1)
    head_i = headti_ref[...]
    h_i = jnp.clip(head_i - 1, 0, _S - 1)[:, None, :]
    v_i = (head_i > 0)[:, None, :]
    c1 = (h_i == jj) & v_i
    c2 = (h_all[None, :, :] == ii) & valid_all[None, :, :]
    a = c1 | c2 | (ii == jj)
    adj_out[...] = a.astype(jnp.float32)
    dep_out[...] = (~a).astype(jnp.int8)
    lrow = jnp.sum((maskt_ref[...] == 0).astype(jnp.int32), axis=0,
                   keepdims=True)
    pad = ~(lax.broadcasted_iota(jnp.int32, (_S, _BL), 0) < lrow)
    pad_out[...] = jnp.broadcast_to(pad.astype(jnp.int8)[None, :, :],
                                    (_SI, _S, _BL))
    seq_out[...] = (~(jj <= ii)).astype(jnp.int8)


_SS = 10
_EBL = 1024


def _emb_body(w_ref, post_ref, nert_ref, fusedt_ref, posemb_ref, g_ref, b_ref,
              emb_out):
    nv = _POS_V + _NER_V
    io = lax.broadcasted_iota(jnp.int32, (nv, _EBL), 0)
    for k in range(_SS):
        w_t = jnp.transpose(w_ref[k], (1, 0))
        p = post_ref[k]
        n = nert_ref[k]
        oh = ((io == p) | (io == n + _POS_V)).astype(jnp.float32)
        pn = jnp.dot(fusedt_ref[...], oh,
                     preferred_element_type=jnp.float32)
        e = jnp.concatenate([w_t, pn], axis=0) + posemb_ref[k]
        mean = jnp.mean(e, axis=0, keepdims=True)
        var = jnp.mean((e - mean) * (e - mean), axis=0, keepdims=True)
        emb_out[k] = (e - mean) * lax.rsqrt(var + 1e-5) * g_ref[...] + b_ref[...]


def kernel(words, masks, pos, ner, deprel, head, subj_pos, obj_pos,
           subj_type, obj_type, emb, pos_emb, ner_emb, position_emb,
           ln_gamma, ln_beta):
    nchunk = _B * _S // (_NW * _CH)
    words_t3d = words.T.reshape(_NW, nchunk, _CH)
    word_rows_t = _sc_gather(emb, words_t3d, _EMB).reshape(_S, _B, _EMB)

    nv = _POS_V + _NER_V
    fused = jnp.zeros((nv, 64), jnp.float32)
    fused = fused.at[:_POS_V, :32].set(pos_emb)
    fused = fused.at[_POS_V:, 32:].set(ner_emb)

    f32 = jnp.float32
    i8 = jnp.int8
    embt = pl.pallas_call(
        _emb_body,
        grid=(_S // _SS, _B // _EBL),
        in_specs=[
            pl.BlockSpec((_SS, _EBL, _EMB), lambda s, b: (s, b, 0)),
            pl.BlockSpec((_SS, 1, _EBL), lambda s, b: (s, 0, b)),
            pl.BlockSpec((_SS, 1, _EBL), lambda s, b: (s, 0, b)),
            pl.BlockSpec((64, nv), lambda s, b: (0, 0)),
            pl.BlockSpec((_SS, _IN, 1), lambda s, b: (s, 0, 0)),
            pl.BlockSpec((_IN, 1), lambda s, b: (0, 0)),
            pl.BlockSpec((_IN, 1), lambda s, b: (0, 0)),
        ],
        out_specs=pl.BlockSpec((_SS, _IN, _EBL), lambda s, b: (s, 0, b)),
        out_shape=jax.ShapeDtypeStruct((_S, _IN, _B), f32),
        compiler_params=pltpu.CompilerParams(
            dimension_semantics=("parallel", "parallel")),
    )(word_rows_t, pos.T[:, None, :], ner.T[:, None, :], fused.T,
      position_emb[:_S][:, :, None], ln_gamma.reshape(_IN, 1),
      ln_beta.reshape(_IN, 1))
    embeddings = jnp.transpose(embt, (2, 0, 1))

    head_t = head.T
    masks_t = masks.T
    colspec = pl.BlockSpec((_S, _BL), lambda i, b: (0, b))
    rowspec = pl.BlockSpec((_SI, _BL), lambda i, b: (i, b))
    outt = lambda dt: pl.BlockSpec((_SI, _S, _BL), lambda i, b: (i, 0, b))
    ssb = lambda dt: jax.ShapeDtypeStruct((_S, _S, _B), dt)
    dep_t, pad_t, seq_t, adj_t = pl.pallas_call(
        _mask_body,
        grid=(_S // _SI, _B // _BL),
        in_specs=[colspec, rowspec, colspec],
        out_specs=(outt(i8), outt(i8), outt(i8), outt(f32)),
        out_shape=(ssb(i8), ssb(i8), ssb(i8), ssb(f32)),
        compiler_params=pltpu.CompilerParams(
            dimension_semantics=("parallel", "parallel")),
    )(head_t, head_t, masks_t)
    tr = lambda x: jnp.transpose(x, (2, 0, 1))
    trb = lambda x: jnp.transpose(x.astype(jnp.bool_), (2, 0, 1))
    return (embeddings, trb(dep_t), trb(pad_t), trb(seq_t), tr(adj_t))


# speedup vs baseline: 1.1861x; 1.0027x over previous
import functools

import jax
import jax.numpy as jnp
from jax import lax
from jax.experimental import pallas as pl
from jax.experimental.pallas import tpu as pltpu
from jax.experimental.pallas import tpu_sc as plsc

_B = 1024
_S = 200
_EMB = 128
_POS_V = 50
_NER_V = 20
_IN = 192
_BB = 8
_NW = 32
_CH = 128


def _sc_gather(table, idx_3d, d):
    nchunk = idx_3d.shape[1]
    tot = _NW * nchunk * _CH
    rows_per_w = nchunk * _CH
    mesh = plsc.VectorSubcoreMesh(core_axis_name="c", subcore_axis_name="s")

    @functools.partial(
        pl.kernel,
        out_type=jax.ShapeDtypeStruct((tot, d), jnp.float32),
        mesh=mesh,
        scratch_types=[
            pltpu.VMEM((nchunk, _CH), jnp.int32),
            pltpu.VMEM((_CH, d), jnp.float32),
            pltpu.VMEM((_CH, d), jnp.float32),
            pltpu.SemaphoreType.DMA,
            pltpu.SemaphoreType.DMA,
        ],
    )
    def gather_kernel(table_hbm, idx_hbm, out_hbm, idx_v, rows0, rows1, sem0, sem1):
        wid = lax.axis_index("s") * 2 + lax.axis_index("c")
        base = wid * rows_per_w
        pltpu.sync_copy(idx_hbm.at[wid], idx_v)
        pltpu.async_copy(table_hbm.at[idx_v.at[0]], rows0, sem0)

        def body(j, carry):
            buf = lax.rem(j, 2)

            @pl.when(j + 1 < nchunk)
            def _():
                @pl.when(buf == 0)
                def _():
                    pltpu.async_copy(table_hbm.at[idx_v.at[j + 1]], rows1, sem1)

                @pl.when(buf == 1)
                def _():
                    pltpu.async_copy(table_hbm.at[idx_v.at[j + 1]], rows0, sem0)

            @pl.when(buf == 0)
            def _():
                pltpu.make_async_copy(table_hbm.at[idx_v.at[0]], rows0, sem0).wait()
                pltpu.sync_copy(rows0, out_hbm.at[pl.ds(base + j * _CH, _CH)])

            @pl.when(buf == 1)
            def _():
                pltpu.make_async_copy(table_hbm.at[idx_v.at[0]], rows1, sem1).wait()
                pltpu.sync_copy(rows1, out_hbm.at[pl.ds(base + j * _CH, _CH)])

            return carry

        lax.fori_loop(0, nchunk, body, 0)

    return gather_kernel(table, idx_3d)


def _tc_body(w_ref, pos_ref, ner_ref, fused_ref, posemb_ref, g_ref, b_ref,
             emb_out):
    nv = _POS_V + _NER_V
    w = w_ref[...]
    p = pos_ref[..., 0][..., None]
    n = ner_ref[..., 0][..., None]
    io = lax.broadcasted_iota(jnp.int32, (_BB, _S, nv), 2)
    oh = ((io == p) | (io == n + _POS_V)).astype(jnp.float32)
    pn = lax.dot_general(oh, fused_ref[...],
                         dimension_numbers=(((2,), (0,)), ((), ())),
                         preferred_element_type=jnp.float32)
    e = jnp.concatenate([w, pn], axis=2)
    e = e + posemb_ref[:_S][None, :, :]
    mean = jnp.mean(e, axis=-1, keepdims=True)
    var = jnp.mean((e - mean) * (e - mean), axis=-1, keepdims=True)
    emb_out[...] = (e - mean) * lax.rsqrt(var + 1e-5) * g_ref[...] + b_ref[...]


_SI = 8
_BL = 1024


def _mask_body(headt_ref, headti_ref, maskt_ref, packed_out, adj_out):
    i0 = pl.program_id(0) * _SI
    head_all = headt_ref[...]
    h_all = jnp.clip(head_all - 1, 0, _S - 1)
    valid_all = head_all > 0
    ii = i0 + lax.broadcasted_iota(jnp.int32, (_SI, _S, _BL), 0)
    jj = lax.broadcasted_iota(jnp.int32, (_SI, _S, _BL), 1)
    head_i = headti_ref[...]
    h_i = jnp.clip(head_i - 1, 0, _S - 1)[:, None, :]
    v_i = (head_i > 0)[:, None, :]
    c1 = (h_i == jj) & v_i
    c2 = (h_all[None, :, :] == ii) & valid_all[None, :, :]
    a = c1 | c2 | (ii == jj)
    adj_out[...] = a.astype(jnp.float32)
    lrow = jnp.sum((maskt_ref[...] == 0).astype(jnp.int32), axis=0,
                   keepdims=True)
    pad = ~(lax.broadcasted_iota(jnp.int32, (_S, _BL), 0) < lrow)
    pad32 = jnp.broadcast_to(pad[None, :, :], (_SI, _S, _BL)).astype(jnp.int32)
    seq32 = (~(jj <= ii)).astype(jnp.int32)
    packed = (~a).astype(jnp.int32) | (pad32 * 2) | (seq32 * 4)
    packed_out[...] = packed.astype(jnp.int8)


_SS = 10
_EBL = 1024


def _emb_body(w_ref, post_ref, nert_ref, fusedt_ref, posemb_ref, g_ref, b_ref,
              emb_out):
    nv = _POS_V + _NER_V
    io = lax.broadcasted_iota(jnp.int32, (nv, _EBL), 0)
    for k in range(_SS):
        w_t = jnp.transpose(w_ref[k], (1, 0))
        p = post_ref[k]
        n = nert_ref[k]
        oh = ((io == p) | (io == n + _POS_V)).astype(jnp.float32)
        pn = jnp.dot(fusedt_ref[...], oh,
                     preferred_element_type=jnp.float32)
        e = jnp.concatenate([w_t, pn], axis=0) + posemb_ref[k]
        mean = jnp.mean(e, axis=0, keepdims=True)
        var = jnp.mean((e - mean) * (e - mean), axis=0, keepdims=True)
        emb_out[k] = (e - mean) * lax.rsqrt(var + 1e-5) * g_ref[...] + b_ref[...]


def kernel(words, masks, pos, ner, deprel, head, subj_pos, obj_pos,
           subj_type, obj_type, emb, pos_emb, ner_emb, position_emb,
           ln_gamma, ln_beta):
    nchunk = _B * _S // (_NW * _CH)
    words_t3d = words.T.reshape(_NW, nchunk, _CH)
    word_rows_t = _sc_gather(emb, words_t3d, _EMB).reshape(_S, _B, _EMB)

    nv = _POS_V + _NER_V
    fused = jnp.zeros((nv, 64), jnp.float32)
    fused = fused.at[:_POS_V, :32].set(pos_emb)
    fused = fused.at[_POS_V:, 32:].set(ner_emb)

    f32 = jnp.float32
    i8 = jnp.int8
    embt = pl.pallas_call(
        _emb_body,
        grid=(_S // _SS, _B // _EBL),
        in_specs=[
            pl.BlockSpec((_SS, _EBL, _EMB), lambda s, b: (s, b, 0)),
            pl.BlockSpec((_SS, 1, _EBL), lambda s, b: (s, 0, b)),
            pl.BlockSpec((_SS, 1, _EBL), lambda s, b: (s, 0, b)),
            pl.BlockSpec((64, nv), lambda s, b: (0, 0)),
            pl.BlockSpec((_SS, _IN, 1), lambda s, b: (s, 0, 0)),
            pl.BlockSpec((_IN, 1), lambda s, b: (0, 0)),
            pl.BlockSpec((_IN, 1), lambda s, b: (0, 0)),
        ],
        out_specs=pl.BlockSpec((_SS, _IN, _EBL), lambda s, b: (s, 0, b)),
        out_shape=jax.ShapeDtypeStruct((_S, _IN, _B), f32),
        compiler_params=pltpu.CompilerParams(
            dimension_semantics=("parallel", "parallel")),
    )(word_rows_t, pos.T[:, None, :], ner.T[:, None, :], fused.T,
      position_emb[:_S][:, :, None], ln_gamma.reshape(_IN, 1),
      ln_beta.reshape(_IN, 1))
    embeddings = jnp.transpose(embt, (2, 0, 1))

    head_t = head.T
    masks_t = masks.T
    colspec = pl.BlockSpec((_S, _BL), lambda i, b: (0, b))
    rowspec = pl.BlockSpec((_SI, _BL), lambda i, b: (i, b))
    outt = lambda dt: pl.BlockSpec((_SI, _S, _BL), lambda i, b: (i, 0, b))
    ssb = lambda dt: jax.ShapeDtypeStruct((_S, _S, _B), dt)
    packed_t, adj_t = pl.pallas_call(
        _mask_body,
        grid=(_S // _SI, _B // _BL),
        in_specs=[colspec, rowspec, colspec],
        out_specs=(outt(i8), outt(f32)),
        out_shape=(ssb(i8), ssb(f32)),
        compiler_params=pltpu.CompilerParams(
            dimension_semantics=("parallel", "parallel")),
    )(head_t, head_t, masks_t)
    tr = lambda x: jnp.transpose(x, (2, 0, 1))
    unpack = lambda bit: jnp.transpose((packed_t & bit).astype(jnp.bool_),
                                       (2, 0, 1))
    return (embeddings, unpack(1), unpack(2), unpack(4), tr(adj_t))
